# Initial kernel scaffold; baseline (speedup 1.0000x reference)
import functools

import jax
import jax.numpy as jnp
from jax import lax
from jax.experimental import pallas as pl
from jax.experimental.pallas import tpu as pltpu
from jax.experimental.pallas import tpu_sc as plsc

N = 10000
E = 320000
D = 128
DH = D // 2
NS = 16
B = 125
W = E // NS // B
ROWS_PER_TILE = N // NS
ZROWS = 125


def _body(x2_hbm, row_hbm, col_hbm, out_hbm,
          x_sh, acc_sh, colbuf, rowbuf, msg, zbuf, gsem0, gsem1):
    c = lax.axis_index("c")
    s = lax.axis_index("s")
    r0 = s * ROWS_PER_TILE

    pltpu.sync_copy(x2_hbm.at[c, pl.ds(r0, ROWS_PER_TILE)],
                    x_sh.at[pl.ds(r0, ROWS_PER_TILE)])

    zeros16 = jnp.zeros((16,), jnp.float32)

    def _zero_row(r, carry):
        for k in range(DH // 16):
            zbuf[r, pl.ds(k * 16, 16)] = zeros16
        return carry

    lax.fori_loop(0, ZROWS, _zero_row, 0)
    for b in range(ROWS_PER_TILE // ZROWS):
        pltpu.sync_copy(zbuf, acc_sh.at[pl.ds(r0 + b * ZROWS, ZROWS)])

    pltpu.sync_copy(col_hbm.at[pl.ds(s * W, W)], colbuf)
    pltpu.sync_copy(row_hbm.at[pl.ds(s * W, W)], rowbuf)

    plsc.subcore_barrier()

    gsems = (gsem0, gsem1)

    def _start_gather(j, b):
        pltpu.async_copy(x_sh.at[colbuf.at[j]], msg.at[b], gsems[b])

    def _wait_gather(j, b):
        pltpu.make_async_copy(x_sh.at[colbuf.at[j]], msg.at[b],
                              gsems[b]).wait()

    def _scatter_add(j, b):
        pltpu.sync_copy(msg.at[b], acc_sh.at[rowbuf.at[j]], add=True)

    for b in range(2):
        _start_gather(b, b)

    def _win_pair(o, carry):
        for b in range(2):
            j = 2 * o + b
            _wait_gather(j, b)
            _scatter_add(j, b)
            _start_gather(j + 2, b)
        return carry

    lax.fori_loop(0, W // 2 - 1, _win_pair, 0)
    for b in range(2):
        j = W - 2 + b
        _wait_gather(j, b)
        _scatter_add(j, b)

    plsc.subcore_barrier()

    pltpu.sync_copy(acc_sh.at[pl.ds(r0, ROWS_PER_TILE)],
                    out_hbm.at[c, pl.ds(r0, ROWS_PER_TILE)])


@jax.jit
def kernel(x, edge_index):
    x2 = jnp.stack([x[:, :DH], x[:, DH:]])
    row2 = edge_index[0].reshape(E // B, B)
    col2 = edge_index[1].reshape(E // B, B)

    mesh = plsc.VectorSubcoreMesh(core_axis_name="c", subcore_axis_name="s")
    out2 = pl.kernel(
        _body,
        out_type=jax.ShapeDtypeStruct((2, N, DH), jnp.float32),
        mesh=mesh,
        scratch_types=[
            pltpu.VMEM_SHARED((N, DH), jnp.float32),
            pltpu.VMEM_SHARED((N, DH), jnp.float32),
            pltpu.VMEM((W, B), jnp.int32),
            pltpu.VMEM((W, B), jnp.int32),
            pltpu.VMEM((2, B, DH), jnp.float32),
            pltpu.VMEM((ZROWS, DH), jnp.float32),
            pltpu.SemaphoreType.DMA,
            pltpu.SemaphoreType.DMA,
        ],
    )(x2, row2, col2)
    return jnp.concatenate([out2[0], out2[1]], axis=1)


# speedup vs baseline: 8.5590x; 8.5590x over previous
import functools

import jax
import jax.numpy as jnp
from jax import lax
from jax.experimental import pallas as pl
from jax.experimental.pallas import tpu as pltpu
from jax.experimental.pallas import tpu_sc as plsc

N = 10000
NP = 10240
E = 320000
D = 128
DH = D // 2
NS = 16
B = 125
W = E // NS // B
CH = 32
NCHUNK = W // CH
ROWS_PER_TILE = NP // NS
ZROWS = 128


def _body(x2_hbm, row_hbm, col_hbm, out_hbm,
          x_sh, acc_sh, colbuf, rowbuf, msg, zbuf, gsem0, gsem1):
    c = lax.axis_index("c")
    s = lax.axis_index("s")
    r0 = s * ROWS_PER_TILE

    pltpu.sync_copy(x2_hbm.at[c, pl.ds(r0, ROWS_PER_TILE)],
                    x_sh.at[pl.ds(r0, ROWS_PER_TILE)])

    zeros16 = jnp.zeros((16,), jnp.float32)

    def _zero_row(r, carry):
        for k in range(DH // 16):
            zbuf[r, pl.ds(k * 16, 16)] = zeros16
        return carry

    lax.fori_loop(0, ZROWS, _zero_row, 0)
    for b in range(ROWS_PER_TILE // ZROWS):
        pltpu.sync_copy(zbuf, acc_sh.at[pl.ds(r0 + b * ZROWS, ZROWS)])

    plsc.subcore_barrier()

    gsems = (gsem0, gsem1)

    def _start_gather(w):
        pltpu.async_copy(x_sh.at[colbuf.at[w]], msg.at[w % 2],
                         gsems[w % 2])

    def _wait_gather(w):
        pltpu.make_async_copy(x_sh.at[colbuf.at[w]], msg.at[w % 2],
                              gsems[w % 2]).wait()

    def _scatter_add(w):
        pltpu.sync_copy(msg.at[w % 2], acc_sh.at[rowbuf.at[w]], add=True)

    def _chunk(k, carry):
        w0 = k * CH
        pltpu.sync_copy(col_hbm.at[pl.ds(s * W + w0, CH)], colbuf)
        pltpu.sync_copy(row_hbm.at[pl.ds(s * W + w0, CH)], rowbuf)
        for w in range(2):
            _start_gather(w)
        for w in range(CH):
            _wait_gather(w)
            _scatter_add(w)
            if w + 2 < CH:
                _start_gather(w + 2)
        return carry

    lax.fori_loop(0, NCHUNK, _chunk, 0)

    plsc.subcore_barrier()

    pltpu.sync_copy(acc_sh.at[pl.ds(r0, ROWS_PER_TILE)],
                    out_hbm.at[c, pl.ds(r0, ROWS_PER_TILE)])


@jax.jit
def kernel(x, edge_index):
    xp = jnp.pad(x, ((0, NP - N), (0, 0)))
    x2 = jnp.stack([xp[:, :DH], xp[:, DH:]])
    row2 = edge_index[0].reshape(E // B, B)
    col2 = edge_index[1].reshape(E // B, B)

    mesh = plsc.VectorSubcoreMesh(core_axis_name="c", subcore_axis_name="s")
    out2 = pl.kernel(
        _body,
        out_type=jax.ShapeDtypeStruct((2, NP, DH), jnp.float32),
        mesh=mesh,
        scratch_types=[
            pltpu.VMEM_SHARED((NP, DH), jnp.float32),
            pltpu.VMEM_SHARED((NP, DH), jnp.float32),
            pltpu.VMEM((CH, B), jnp.int32),
            pltpu.VMEM((CH, B), jnp.int32),
            pltpu.VMEM((2, B, DH), jnp.float32),
            pltpu.VMEM((ZROWS, DH), jnp.float32),
            pltpu.SemaphoreType.DMA,
            pltpu.SemaphoreType.DMA,
        ],
        compiler_params=pltpu.CompilerParams(use_tc_tiling_on_sc=False),
    )(x2, row2, col2)
    return jnp.concatenate([out2[0, :N], out2[1, :N]], axis=1)


# speedup vs baseline: 10.6797x; 1.2478x over previous
import jax
import jax.numpy as jnp
from jax import lax
from jax.experimental import pallas as pl
from jax.experimental.pallas import tpu as pltpu
from jax.experimental.pallas import tpu_sc as plsc

N = 10000
E = 320000
D = 128
DH = D // 2
NS = 16
B = 125
W = E // NS // B
CH = 32
NCHUNK = W // CH
ROWS_PER_TILE = N // NS
ZROWS = 125


def _body(x_hbm, ei_hbm, out_hbm,
          x_sh, acc_sh, colbuf, rowbuf, msg, zbuf, gsem0, gsem1):
    c = lax.axis_index("c")
    s = lax.axis_index("s")
    r0 = s * ROWS_PER_TILE
    c0 = c * DH

    pltpu.sync_copy(x_hbm.at[pl.ds(r0, ROWS_PER_TILE), pl.ds(c0, DH)],
                    x_sh.at[pl.ds(r0, ROWS_PER_TILE)])

    zeros16 = jnp.zeros((16,), jnp.float32)

    def _zero_row(r, carry):
        for k in range(DH // 16):
            zbuf[r, pl.ds(k * 16, 16)] = zeros16
        return carry

    lax.fori_loop(0, ZROWS, _zero_row, 0)
    for b in range(ROWS_PER_TILE // ZROWS):
        pltpu.sync_copy(zbuf, acc_sh.at[pl.ds(r0 + b * ZROWS, ZROWS)])

    plsc.subcore_barrier()

    gsems = (gsem0, gsem1)

    def _start_gather(w):
        pltpu.async_copy(x_sh.at[colbuf.at[w]], msg.at[w % 2],
                         gsems[w % 2])

    def _wait_gather(w):
        pltpu.make_async_copy(x_sh.at[colbuf.at[w]], msg.at[w % 2],
                              gsems[w % 2]).wait()

    def _scatter_add(w):
        pltpu.sync_copy(msg.at[w % 2], acc_sh.at[rowbuf.at[w]], add=True)

    def _chunk(k, carry):
        w0 = k * CH
        pltpu.sync_copy(ei_hbm.at[1, pl.ds(s * W + w0, CH)], colbuf)
        pltpu.sync_copy(ei_hbm.at[0, pl.ds(s * W + w0, CH)], rowbuf)
        for w in range(2):
            _start_gather(w)
        for w in range(CH):
            _wait_gather(w)
            _scatter_add(w)
            if w + 2 < CH:
                _start_gather(w + 2)
        return carry

    lax.fori_loop(0, NCHUNK, _chunk, 0)

    plsc.subcore_barrier()

    pltpu.sync_copy(acc_sh.at[pl.ds(r0, ROWS_PER_TILE)],
                    out_hbm.at[pl.ds(r0, ROWS_PER_TILE), pl.ds(c0, DH)])


@jax.jit
def kernel(x, edge_index):
    ei3 = edge_index.reshape(2, E // B, B)

    mesh = plsc.VectorSubcoreMesh(core_axis_name="c", subcore_axis_name="s")
    out = pl.kernel(
        _body,
        out_type=jax.ShapeDtypeStruct((N, D), jnp.float32),
        mesh=mesh,
        scratch_types=[
            pltpu.VMEM_SHARED((N, DH), jnp.float32),
            pltpu.VMEM_SHARED((N, DH), jnp.float32),
            pltpu.VMEM((CH, B), jnp.int32),
            pltpu.VMEM((CH, B), jnp.int32),
            pltpu.VMEM((2, B, DH), jnp.float32),
            pltpu.VMEM((ZROWS, DH), jnp.float32),
            pltpu.SemaphoreType.DMA,
            pltpu.SemaphoreType.DMA,
        ],
        compiler_params=pltpu.CompilerParams(use_tc_tiling_on_sc=False),
    )(x, ei3)
    return out


# speedup vs baseline: 10.7744x; 1.0089x over previous
import jax
import jax.numpy as jnp
from jax import lax
from jax.experimental import pallas as pl
from jax.experimental.pallas import tpu as pltpu
from jax.experimental.pallas import tpu_sc as plsc

N = 10000
E = 320000
D = 128
DH = D // 2
NS = 16
B = 125
W = E // NS // B
ROWS_PER_TILE = N // NS
ZROWS = 125


def _body(x2_hbm, col_hbm, row_hbm, out_hbm,
          acc_sh, colbuf, rowbuf, msg, zbuf, gsem0, gsem1):
    c = lax.axis_index("c")
    s = lax.axis_index("s")
    r0 = s * ROWS_PER_TILE
    c0 = c * DH

    zeros16 = jnp.zeros((16,), jnp.float32)

    def _zero_row(r, carry):
        for k in range(DH // 16):
            zbuf[r, pl.ds(k * 16, 16)] = zeros16
        return carry

    lax.fori_loop(0, ZROWS, _zero_row, 0)
    for b in range(ROWS_PER_TILE // ZROWS):
        pltpu.sync_copy(zbuf, acc_sh.at[pl.ds(r0 + b * ZROWS, ZROWS)])

    pltpu.sync_copy(col_hbm.at[c, pl.ds(s * W, W)], colbuf)
    pltpu.sync_copy(row_hbm.at[pl.ds(s * W, W)], rowbuf)

    plsc.subcore_barrier()

    gsems = (gsem0, gsem1)

    def _start_gather(w, b):
        pltpu.async_copy(x2_hbm.at[colbuf.at[w]], msg.at[b], gsems[b])

    def _wait_gather(w, b):
        pltpu.make_async_copy(x2_hbm.at[colbuf.at[w]], msg.at[b],
                              gsems[b]).wait()

    def _scatter_add(w, b):
        pltpu.sync_copy(msg.at[b], acc_sh.at[rowbuf.at[w]], add=True)

    for b in range(2):
        _start_gather(b, b)

    def _win_pair(o, carry):
        for b in range(2):
            w = 2 * o + b
            _wait_gather(w, b)
            _scatter_add(w, b)
            _start_gather(w + 2, b)
        return carry

    lax.fori_loop(0, W // 2 - 1, _win_pair, 0)
    for b in range(2):
        w = W - 2 + b
        _wait_gather(w, b)
        _scatter_add(w, b)

    plsc.subcore_barrier()

    pltpu.sync_copy(acc_sh.at[pl.ds(r0, ROWS_PER_TILE)],
                    out_hbm.at[pl.ds(r0, ROWS_PER_TILE), pl.ds(c0, DH)])


@jax.jit
def kernel(x, edge_index):
    x2 = jnp.concatenate([x[:, :DH], x[:, DH:]], axis=0)
    col2 = edge_index[1].reshape(E // B, B)
    col3 = jnp.stack([col2, col2 + N])
    row2 = edge_index[0].reshape(E // B, B)

    mesh = plsc.VectorSubcoreMesh(core_axis_name="c", subcore_axis_name="s")
    out = pl.kernel(
        _body,
        out_type=jax.ShapeDtypeStruct((N, D), jnp.float32),
        mesh=mesh,
        scratch_types=[
            pltpu.VMEM_SHARED((N, DH), jnp.float32),
            pltpu.VMEM((W, B), jnp.int32),
            pltpu.VMEM((W, B), jnp.int32),
            pltpu.VMEM((2, B, DH), jnp.float32),
            pltpu.VMEM((ZROWS, DH), jnp.float32),
            pltpu.SemaphoreType.DMA,
            pltpu.SemaphoreType.DMA,
        ],
        compiler_params=pltpu.CompilerParams(use_tc_tiling_on_sc=False),
    )(x2, col3, row2)
    return out


# speedup vs baseline: 11.2234x; 1.0417x over previous
import jax
import jax.numpy as jnp
from jax import lax
from jax.experimental import pallas as pl
from jax.experimental.pallas import tpu as pltpu
from jax.experimental.pallas import tpu_sc as plsc

N = 10000
E = 320000
D = 128
DH = D // 2
NS = 16
B = 125
W = E // NS // B
ROWS_PER_TILE = N // NS
ZROWS = 125


def _body(x2_hbm, col_hbm, row_hbm, out_hbm,
          acc_sh, colbuf, rowbuf, msg, zbuf,
          gsem0, gsem1, gsem2, gsem3, ssem0, ssem1, ssem2, ssem3):
    c = lax.axis_index("c")
    s = lax.axis_index("s")
    r0 = s * ROWS_PER_TILE
    c0 = c * DH

    zeros16 = jnp.zeros((16,), jnp.float32)

    def _zero_row(r, carry):
        for k in range(DH // 16):
            zbuf[r, pl.ds(k * 16, 16)] = zeros16
        return carry

    lax.fori_loop(0, ZROWS, _zero_row, 0)
    for b in range(ROWS_PER_TILE // ZROWS):
        pltpu.sync_copy(zbuf, acc_sh.at[pl.ds(r0 + b * ZROWS, ZROWS)])

    pltpu.sync_copy(col_hbm.at[c, pl.ds(s * W, W)], colbuf)
    pltpu.sync_copy(row_hbm.at[pl.ds(s * W, W)], rowbuf)

    plsc.subcore_barrier()

    gsems = (gsem0, gsem1, gsem2, gsem3)
    ssems = (ssem0, ssem1, ssem2, ssem3)

    def _start_gather(w, q):
        pltpu.async_copy(x2_hbm.at[colbuf.at[w]], msg.at[q], gsems[q])

    def _wait_gather(w, q):
        pltpu.make_async_copy(x2_hbm.at[colbuf.at[w]], msg.at[q],
                              gsems[q]).wait()

    def _start_scatter(w, q):
        pltpu.async_copy(msg.at[q], acc_sh.at[rowbuf.at[w]], ssems[q],
                         add=True)

    def _wait_scatter(w, q):
        pltpu.make_async_copy(msg.at[q], acc_sh.at[rowbuf.at[w]],
                              ssems[q]).wait()

    for w in range(2):
        _start_gather(w, w % 4)
    for w in range(2):
        _wait_gather(w, w % 4)
        _start_scatter(w, w % 4)
        _start_gather(w + 2, (w + 2) % 4)

    def _quad(o, carry):
        wbase = 4 * o + 2
        for i in range(4):
            w = wbase + i
            q = (2 + i) % 4
            qn = i % 4
            _wait_gather(w, q)
            _start_scatter(w, q)
            _wait_scatter(w - 2, qn)
            _start_gather(w + 2, qn)
        return carry

    lax.fori_loop(0, (W - 4) // 4, _quad, 0)
    for w in range(W - 2, W):
        q = w % 4
        _wait_gather(w, q)
        _start_scatter(w, q)
        _wait_scatter(w - 2, (w - 2) % 4)
    for w in range(W - 2, W):
        _wait_scatter(w, w % 4)

    plsc.subcore_barrier()

    pltpu.sync_copy(acc_sh.at[pl.ds(r0, ROWS_PER_TILE)],
                    out_hbm.at[pl.ds(r0, ROWS_PER_TILE), pl.ds(c0, DH)])


@jax.jit
def kernel(x, edge_index):
    x2 = jnp.concatenate([x[:, :DH], x[:, DH:]], axis=0)
    col2 = edge_index[1].reshape(E // B, B)
    col3 = jnp.stack([col2, col2 + N])
    row2 = edge_index[0].reshape(E // B, B)

    mesh = plsc.VectorSubcoreMesh(core_axis_name="c", subcore_axis_name="s")
    out = pl.kernel(
        _body,
        out_type=jax.ShapeDtypeStruct((N, D), jnp.float32),
        mesh=mesh,
        scratch_types=[
            pltpu.VMEM_SHARED((N, DH), jnp.float32),
            pltpu.VMEM((W, B), jnp.int32),
            pltpu.VMEM((W, B), jnp.int32),
            pltpu.VMEM((4, B, DH), jnp.float32),
            pltpu.VMEM((ZROWS, DH), jnp.float32),
            pltpu.SemaphoreType.DMA,
            pltpu.SemaphoreType.DMA,
            pltpu.SemaphoreType.DMA,
            pltpu.SemaphoreType.DMA,
            pltpu.SemaphoreType.DMA,
            pltpu.SemaphoreType.DMA,
            pltpu.SemaphoreType.DMA,
            pltpu.SemaphoreType.DMA,
        ],
        compiler_params=pltpu.CompilerParams(use_tc_tiling_on_sc=False),
    )(x2, col3, row2)
    return out


# speedup vs baseline: 11.9462x; 1.0644x over previous
import jax
import jax.numpy as jnp
from jax import lax
from jax.experimental import pallas as pl
from jax.experimental.pallas import tpu as pltpu
from jax.experimental.pallas import tpu_sc as plsc

N = 10000
E = 320000
D = 128
DH = D // 2
NS = 16
B = 125
W = E // NS // B
NBUF = 6
ROWS_PER_TILE = N // NS
ZROWS = 125
WMAIN = ((W - 3 - 4) // NBUF) * NBUF


def _body(x2_hbm, col_hbm, row_hbm, out_hbm,
          acc_sh, colbuf, rowbuf, msg,
          gsem0, gsem1, gsem2, gsem3, gsem4, gsem5,
          ssem0, ssem1, ssem2, ssem3, ssem4, ssem5):
    c = lax.axis_index("c")
    s = lax.axis_index("s")
    r0 = s * ROWS_PER_TILE
    c0 = c * DH

    zeros16 = jnp.zeros((16,), jnp.float32)

    def _zero_row(r, carry):
        for k in range(DH // 16):
            msg[0, r, pl.ds(k * 16, 16)] = zeros16
        return carry

    lax.fori_loop(0, ZROWS, _zero_row, 0)
    for b in range(ROWS_PER_TILE // ZROWS):
        pltpu.sync_copy(msg.at[0], acc_sh.at[pl.ds(r0 + b * ZROWS, ZROWS)])

    pltpu.sync_copy(col_hbm.at[c, pl.ds(s * W, W)], colbuf)
    pltpu.sync_copy(row_hbm.at[pl.ds(s * W, W)], rowbuf)

    plsc.subcore_barrier()

    gsems = (gsem0, gsem1, gsem2, gsem3, gsem4, gsem5)
    ssems = (ssem0, ssem1, ssem2, ssem3, ssem4, ssem5)

    def _start_gather(w, q):
        pltpu.async_copy(x2_hbm.at[colbuf.at[w]], msg.at[q], gsems[q])

    def _wait_gather(w, q):
        pltpu.make_async_copy(x2_hbm.at[colbuf.at[w]], msg.at[q],
                              gsems[q]).wait()

    def _start_scatter(w, q):
        pltpu.async_copy(msg.at[q], acc_sh.at[rowbuf.at[w]], ssems[q],
                         add=True)

    def _wait_scatter(w, q):
        pltpu.make_async_copy(msg.at[q], acc_sh.at[rowbuf.at[w]],
                              ssems[q]).wait()

    for w in range(3):
        _start_gather(w, w % NBUF)
    for w in range(3):
        _wait_gather(w, w % NBUF)
        _start_scatter(w, w % NBUF)
        _start_gather(w + 3, (w + 3) % NBUF)

    def _hex(o, carry):
        wbase = NBUF * o + 3
        for i in range(NBUF):
            w = wbase + i
            q = (3 + i) % NBUF
            qn = i % NBUF
            _wait_gather(w, q)
            _start_scatter(w, q)
            _wait_scatter(w - 3, qn)
            _start_gather(w + 3, qn)
        return carry

    lax.fori_loop(0, WMAIN // NBUF, _hex, 0)
    for w in range(3 + WMAIN, W - 3):
        _wait_gather(w, w % NBUF)
        _start_scatter(w, w % NBUF)
        _wait_scatter(w - 3, (w - 3) % NBUF)
        _start_gather(w + 3, (w + 3) % NBUF)
    for w in range(W - 3, W):
        _wait_gather(w, w % NBUF)
        _start_scatter(w, w % NBUF)
        _wait_scatter(w - 3, (w - 3) % NBUF)
    for w in range(W - 3, W):
        _wait_scatter(w, w % NBUF)

    plsc.subcore_barrier()

    pltpu.sync_copy(acc_sh.at[pl.ds(r0, ROWS_PER_TILE)],
                    out_hbm.at[pl.ds(r0, ROWS_PER_TILE), pl.ds(c0, DH)])


@jax.jit
def kernel(x, edge_index):
    x2 = jnp.concatenate([x[:, :DH], x[:, DH:]], axis=0)
    col2 = edge_index[1].reshape(E // B, B)
    col3 = jnp.stack([col2, col2 + N])
    row2 = edge_index[0].reshape(E // B, B)

    mesh = plsc.VectorSubcoreMesh(core_axis_name="c", subcore_axis_name="s")
    out = pl.kernel(
        _body,
        out_type=jax.ShapeDtypeStruct((N, D), jnp.float32),
        mesh=mesh,
        scratch_types=[
            pltpu.VMEM_SHARED((N, DH), jnp.float32),
            pltpu.VMEM((W, B), jnp.int32),
            pltpu.VMEM((W, B), jnp.int32),
            pltpu.VMEM((NBUF, B, DH), jnp.float32),
            pltpu.SemaphoreType.DMA,
            pltpu.SemaphoreType.DMA,
            pltpu.SemaphoreType.DMA,
            pltpu.SemaphoreType.DMA,
            pltpu.SemaphoreType.DMA,
            pltpu.SemaphoreType.DMA,
            pltpu.SemaphoreType.DMA,
            pltpu.SemaphoreType.DMA,
            pltpu.SemaphoreType.DMA,
            pltpu.SemaphoreType.DMA,
            pltpu.SemaphoreType.DMA,
            pltpu.SemaphoreType.DMA,
        ],
        compiler_params=pltpu.CompilerParams(use_tc_tiling_on_sc=False),
    )(x2, col3, row2)
    return out
